# Initial kernel scaffold; baseline (speedup 1.0000x reference)
import jax
import jax.numpy as jnp
from jax.experimental import pallas as pl


def kernel(features, neighbour_distances, edge_index, W_lin, Wf1, bf1, Wf2, bf2, Wm1, bm1, Wm2, bm2):
    raise NotImplementedError("write your pallas kernel here")


# speedup vs baseline: 2.4079x; 2.4079x over previous
import functools
import math

import jax
import jax.numpy as jnp
import numpy as np
from jax import lax
from jax.experimental import pallas as pl
from jax.experimental.pallas import tpu as pltpu
from jax.experimental.pallas import tpu_sc as plsc

LOG2 = float(np.log(2.0))
CUTOFF = 5.0
NC = 2
NS = 16
NW = NC * NS
BLK = 128


def _ssp(x):
    return jax.nn.softplus(x) - LOG2


def _h_body(x_ref, w_ref, o_ref):
    o_ref[...] = jnp.dot(x_ref[...], w_ref[...],
                         preferred_element_type=jnp.float32)


def _ef_body(d_ref, wf1_ref, bf1_ref, wf2_ref, bf2_ref, o_ref, *, n_feat):
    d = d_ref[...]
    delta = CUTOFF / (n_feat - 1)
    coef = -1.0 / (2.0 * delta * delta)
    centers = (lax.broadcasted_iota(jnp.int32, (1, n_feat), 1)
               .astype(jnp.float32) * delta)
    g = jnp.exp(coef * (d - centers) ** 2)
    x = _ssp(jnp.dot(g, wf1_ref[...], preferred_element_type=jnp.float32)
             + bf1_ref[...])
    o_ref[...] = (jnp.dot(x, wf2_ref[...], preferred_element_type=jnp.float32)
                  + bf2_ref[...])


def _out_body(p_ref, wm1_ref, bm1_ref, wm2_ref, bm2_ref, o_ref):
    x = p_ref[0] + p_ref[1]
    a = _ssp(jnp.dot(x, wm1_ref[...], preferred_element_type=jnp.float32)
             + bm1_ref[...])
    o_ref[...] = (jnp.dot(a, wm2_ref[...], preferred_element_type=jnp.float32)
                  + bm2_ref[...])


def _sc_body(h_hbm, ef_hbm, src_hbm, dst_hbm, out_hbm,
             idx_src_v, idx_dst_v, rows_v, ef_v, agg_sh, sem1, sem2,
             *, per_w_blks, rows_per_s, n_vregs):
    c = lax.axis_index("c")
    s = lax.axis_index("s")
    wid = s * NC + c

    def zero_body(i, _):
        for v in range(n_vregs):
            rows_v[i, pl.ds(v * 16, 16)] = jnp.zeros((16,), jnp.float32)
        return _
    lax.fori_loop(0, BLK, zero_body, 0)

    base_row = s * rows_per_s
    for k in range(rows_per_s // BLK):
        pltpu.sync_copy(rows_v, agg_sh.at[pl.ds(base_row + k * BLK, BLK)])
    plsc.subcore_barrier()

    e0 = wid * (per_w_blks * BLK)

    def blk_body(b, _):
        base = e0 + b * BLK
        pltpu.sync_copy(src_hbm.at[pl.ds(base, BLK)], idx_src_v)
        pltpu.sync_copy(dst_hbm.at[pl.ds(base, BLK)], idx_dst_v)
        cp1 = pltpu.async_copy(h_hbm.at[idx_src_v], rows_v, sem1)
        cp2 = pltpu.async_copy(ef_hbm.at[pl.ds(base, BLK)], ef_v, sem2)
        cp1.wait()
        cp2.wait()

        def mul_body(i, carry):
            for v in range(n_vregs):
                sl = pl.ds(v * 16, 16)
                rows_v[i, sl] = rows_v[i, sl] * ef_v[i, sl]
            return carry
        lax.fori_loop(0, BLK, mul_body, 0)

        pltpu.sync_copy(rows_v, agg_sh.at[idx_dst_v], add=True)
        return _
    lax.fori_loop(0, per_w_blks, blk_body, 0)

    plsc.subcore_barrier()
    pltpu.sync_copy(agg_sh.at[pl.ds(base_row, rows_per_s)],
                    out_hbm.at[c, pl.ds(base_row, rows_per_s)])


def kernel(features, neighbour_distances, edge_index, W_lin,
           Wf1, bf1, Wf2, bf2, Wm1, bm1, Wm2, bm2):
    N, C = features.shape
    E = neighbour_distances.shape[0]
    F = Wf1.shape[0]
    n_vregs = C // 16

    per_w_blks = -(-E // (NW * BLK))
    e_pad = per_w_blks * BLK * NW
    pad = e_pad - E
    rows_per_s = -(-N // (NS * BLK)) * BLK
    n_pad = rows_per_s * NS

    src = jnp.concatenate([edge_index[1],
                           jnp.zeros((pad,), jnp.int32)])
    dst = jnp.concatenate([edge_index[0],
                           jnp.full((pad,), N, jnp.int32)])
    d_pad = jnp.concatenate([neighbour_distances,
                             jnp.zeros((pad,), jnp.float32)]).reshape(e_pad, 1)

    nblk = 2000
    h = pl.pallas_call(
        _h_body,
        grid=(N // nblk,),
        in_specs=[pl.BlockSpec((nblk, C), lambda i: (i, 0)),
                  pl.BlockSpec((C, C), lambda i: (0, 0))],
        out_specs=pl.BlockSpec((nblk, C), lambda i: (i, 0)),
        out_shape=jax.ShapeDtypeStruct((N, C), jnp.float32),
    )(features, W_lin)

    eblk = 4096
    ef = pl.pallas_call(
        functools.partial(_ef_body, n_feat=F),
        grid=(e_pad // eblk,),
        in_specs=[pl.BlockSpec((eblk, 1), lambda i: (i, 0)),
                  pl.BlockSpec((F, C), lambda i: (0, 0)),
                  pl.BlockSpec((1, C), lambda i: (0, 0)),
                  pl.BlockSpec((C, C), lambda i: (0, 0)),
                  pl.BlockSpec((1, C), lambda i: (0, 0))],
        out_specs=pl.BlockSpec((eblk, C), lambda i: (i, 0)),
        out_shape=jax.ShapeDtypeStruct((e_pad, C), jnp.float32),
    )(d_pad, Wf1, bf1.reshape(1, C), Wf2, bf2.reshape(1, C))

    sc_call = pl.kernel(
        functools.partial(_sc_body, per_w_blks=per_w_blks,
                          rows_per_s=rows_per_s, n_vregs=n_vregs),
        out_type=jax.ShapeDtypeStruct((NC, n_pad, C), jnp.float32),
        scratch_types=[
            pltpu.VMEM((BLK,), jnp.int32),
            pltpu.VMEM((BLK,), jnp.int32),
            pltpu.VMEM((BLK, C), jnp.float32),
            pltpu.VMEM((BLK, C), jnp.float32),
            pltpu.VMEM_SHARED((n_pad, C), jnp.float32),
            pltpu.SemaphoreType.DMA,
            pltpu.SemaphoreType.DMA,
        ],
        mesh=plsc.VectorSubcoreMesh(core_axis_name="c", subcore_axis_name="s"),
    )
    partials = sc_call(h, ef, src, dst)

    out = pl.pallas_call(
        _out_body,
        grid=(N // nblk,),
        in_specs=[pl.BlockSpec((NC, nblk, C), lambda i: (0, i, 0)),
                  pl.BlockSpec((C, C), lambda i: (0, 0)),
                  pl.BlockSpec((1, C), lambda i: (0, 0)),
                  pl.BlockSpec((C, C), lambda i: (0, 0)),
                  pl.BlockSpec((1, C), lambda i: (0, 0))],
        out_specs=pl.BlockSpec((nblk, C), lambda i: (i, 0)),
        out_shape=jax.ShapeDtypeStruct((N, C), jnp.float32),
    )(partials, Wm1, bm1.reshape(1, C), Wm2, bm2.reshape(1, C))
    return out
